# baseline (speedup 1.0000x reference)
import functools

import jax
import jax.numpy as jnp
from jax.experimental import pallas as pl
from jax.experimental.pallas import tpu as pltpu

_Ns = 1000
_bd = 20.0
_bm = 0.1

B = 256
F = 3 * 64 * 64
BB = 8


def _tables():
    ts = jnp.linspace(1e-05, 1.0, _Ns, dtype=jnp.float32)
    betas = (_bm + (_bd - _bm) * ts) / _Ns
    alphas = (1.0 - betas).astype(jnp.float32)
    acp = jnp.cumprod(alphas)
    acp_prev = jnp.concatenate([jnp.ones((1,), jnp.float32), acp[:-1]])
    pv = betas * (1.0 - acp_prev) / (1.0 - acp)
    plv = jnp.log(jnp.clip(pv, 1e-20, None))
    c1 = betas * jnp.sqrt(acp_prev) / (1.0 - acp)
    c2 = (1.0 - acp_prev) * jnp.sqrt(alphas) / (1.0 - acp)
    return (pv.astype(jnp.float32), plv.astype(jnp.float32),
            c1.astype(jnp.float32), c2.astype(jnp.float32))


def _body(i_ref, pv_ref, plv_ref, c1_ref, c2_ref, x0_ref, xi_ref,
          mean_ref, var_ref):
    b = pl.program_id(0)
    for r in range(BB):
        t = i_ref[b * BB + r]
        c1v = c1_ref[t]
        c2v = c2_ref[t]
        mean_ref[r, :] = c1v * x0_ref[r, :] + c2v * xi_ref[r, :]
        var_ref[r, :] = jnp.concatenate(
            [jnp.full((64,), pv_ref[t], jnp.float32),
             jnp.full((64,), plv_ref[t], jnp.float32)])


@jax.jit
def kernel(x0, x_i, i):
    pv, plv, c1, c2 = _tables()
    x0r = x0.reshape(B, F)
    xir = x_i.reshape(B, F)
    grid = (B // BB,)
    smem = pl.BlockSpec(memory_space=pltpu.SMEM)
    mean, var = pl.pallas_call(
        _body,
        grid=grid,
        in_specs=[smem, smem, smem, smem, smem,
                  pl.BlockSpec((BB, F), lambda b: (b, 0)),
                  pl.BlockSpec((BB, F), lambda b: (b, 0))],
        out_specs=[pl.BlockSpec((BB, F), lambda b: (b, 0)),
                   pl.BlockSpec((BB, 128), lambda b: (b, 0))],
        out_shape=[jax.ShapeDtypeStruct((B, F), jnp.float32),
                   jax.ShapeDtypeStruct((B, 128), jnp.float32)],
    )(i, pv, plv, c1, c2, x0r, xir)
    posterior_mean = mean.reshape(x0.shape)
    posterior_variance = var[:, 0].reshape(B, 1, 1, 1)
    posterior_log_variance_clipped = var[:, 64].reshape(B, 1, 1, 1)
    return (posterior_mean, posterior_variance,
            posterior_log_variance_clipped)


# speedup vs baseline: 1.1902x; 1.1902x over previous
import functools

import jax
import jax.numpy as jnp
from jax.experimental import pallas as pl
from jax.experimental.pallas import tpu as pltpu

_Ns = 1000
_bd = 20.0
_bm = 0.1

B = 256
F = 3 * 64 * 64
BB = 32


def _tables():
    ts = jnp.linspace(1e-05, 1.0, _Ns, dtype=jnp.float32)
    betas = (_bm + (_bd - _bm) * ts) / _Ns
    alphas = (1.0 - betas).astype(jnp.float32)
    acp = jnp.cumprod(alphas)
    acp_prev = jnp.concatenate([jnp.ones((1,), jnp.float32), acp[:-1]])
    pv = betas * (1.0 - acp_prev) / (1.0 - acp)
    plv = jnp.log(jnp.clip(pv, 1e-20, None))
    c1 = betas * jnp.sqrt(acp_prev) / (1.0 - acp)
    c2 = (1.0 - acp_prev) * jnp.sqrt(alphas) / (1.0 - acp)
    return (pv.astype(jnp.float32), plv.astype(jnp.float32),
            c1.astype(jnp.float32), c2.astype(jnp.float32))


def _body(i_ref, pv_ref, plv_ref, c1_ref, c2_ref, x0_ref, xi_ref,
          mean_ref, var_ref):
    b = pl.program_id(0)
    for r in range(BB):
        t = i_ref[b * BB + r]
        c1v = c1_ref[t]
        c2v = c2_ref[t]
        mean_ref[r, :] = c1v * x0_ref[r, :] + c2v * xi_ref[r, :]
        var_ref[r, :] = jnp.concatenate(
            [jnp.full((64,), pv_ref[t], jnp.float32),
             jnp.full((64,), plv_ref[t], jnp.float32)])


@jax.jit
def kernel(x0, x_i, i):
    pv, plv, c1, c2 = _tables()
    x0r = x0.reshape(B, F)
    xir = x_i.reshape(B, F)
    grid = (B // BB,)
    smem = pl.BlockSpec(memory_space=pltpu.SMEM)
    mean, var = pl.pallas_call(
        _body,
        grid=grid,
        in_specs=[smem, smem, smem, smem, smem,
                  pl.BlockSpec((BB, F), lambda b: (b, 0)),
                  pl.BlockSpec((BB, F), lambda b: (b, 0))],
        out_specs=[pl.BlockSpec((BB, F), lambda b: (b, 0)),
                   pl.BlockSpec((BB, 128), lambda b: (b, 0))],
        out_shape=[jax.ShapeDtypeStruct((B, F), jnp.float32),
                   jax.ShapeDtypeStruct((B, 128), jnp.float32)],
    )(i, pv, plv, c1, c2, x0r, xir)
    posterior_mean = mean.reshape(x0.shape)
    posterior_variance = var[:, 0].reshape(B, 1, 1, 1)
    posterior_log_variance_clipped = var[:, 64].reshape(B, 1, 1, 1)
    return (posterior_mean, posterior_variance,
            posterior_log_variance_clipped)
